# Initial kernel scaffold; baseline (speedup 1.0000x reference)
import functools

import numpy as np
import jax
import jax.numpy as jnp
from jax import lax
from jax.experimental import pallas as pl
from jax.experimental.pallas import tpu as pltpu
from jax.experimental.pallas import tpu_sc as plsc

N = 10000
DIN = 128
DH = 256
DHF = 128
NG = 64
EPS = 1e-5
NROW = 80
NPAD = NROW * 128
CHUNK = 128
NCH = 81
E2P = 32 * NCH * CHUNK
RPS = N // 16
NBLK = 1000
GRID = N // NBLK

_MESH = plsc.VectorSubcoreMesh(core_axis_name="c", subcore_axis_name="s")
_SCP = pltpu.CompilerParams(needs_layout_passes=False)
_HI = lax.Precision.HIGHEST
_BN0 = np.float32(1.0 / np.sqrt(1.0 + EPS))


def _clean(w):
    w = jnp.where(w != w, 0.0, w)
    return jnp.where(jnp.abs(w) == jnp.inf, 0.0, w)


@functools.partial(
    pl.kernel,
    out_type=jax.ShapeDtypeStruct((2, NROW, 128), jnp.float32),
    mesh=_MESH,
    compiler_params=_SCP,
    scratch_types=[
        pltpu.VMEM((CHUNK,), jnp.int32),
        pltpu.VMEM((CHUNK,), jnp.float32),
        pltpu.VMEM((NROW, 128), jnp.float32),
        pltpu.VMEM((NROW,), jnp.int32),
        pltpu.VMEM_SHARED((NROW, 128), jnp.float32),
    ],
)
def _deg_kernel(col_hbm, ew_hbm, out_hbm, col_v, ew_v, acc_v, idx_v, acc_sh):
    c = lax.axis_index("c")
    s = lax.axis_index("s")
    w = s * 2 + c

    def zrow(i, _):
        for f in range(8):
            acc_v[i, pl.ds(f * 16, 16)] = jnp.zeros((16,), jnp.float32)
        return 0
    lax.fori_loop(0, NROW, zrow, 0)
    pltpu.sync_copy(acc_v.at[pl.ds(0, 5)], acc_sh.at[pl.ds(s * 5, 5)])

    def mkidx(k, _):
        idx_v[pl.ds(k * 16, 16)] = lax.iota(jnp.int32, 16) + k * 16
        return 0
    lax.fori_loop(0, NROW // 16, mkidx, 0)
    plsc.subcore_barrier()

    def chunk(j, _):
        pltpu.sync_copy(col_hbm.at[w, j], col_v)
        pltpu.sync_copy(ew_hbm.at[w, j], ew_v)

        def grp(k, _):
            cv = col_v[pl.ds(k * 16, 16)]
            wv = _clean(ew_v[pl.ds(k * 16, 16)])
            i0 = lax.shift_right_logical(cv, 7)
            i1 = lax.bitwise_and(cv, 127)
            plsc.addupdate_scatter(acc_v, [i0, i1], wv)
            return 0
        lax.fori_loop(0, CHUNK // 16, grp, 0)
        return 0
    lax.fori_loop(0, NCH, chunk, 0)

    pltpu.sync_copy(acc_v, acc_sh.at[idx_v], add=True)
    plsc.subcore_barrier()
    pltpu.sync_copy(acc_sh.at[pl.ds(s * 5, 5)], out_hbm.at[c, pl.ds(s * 5, 5)])


@functools.partial(
    pl.kernel,
    out_type=(
        jax.ShapeDtypeStruct((2, N, DIN), jnp.float32),
        jax.ShapeDtypeStruct((32, NCH, CHUNK), jnp.float32),
    ),
    mesh=_MESH,
    compiler_params=_SCP,
    scratch_types=[
        pltpu.VMEM((NPAD,), jnp.float32),
        pltpu.VMEM((CHUNK,), jnp.int32),
        pltpu.VMEM((CHUNK,), jnp.int32),
        pltpu.VMEM((CHUNK,), jnp.float32),
        pltpu.VMEM((CHUNK, DIN), jnp.float32),
        pltpu.VMEM_SHARED((N, DIN), jnp.float32),
        pltpu.SemaphoreType.DMA,
    ],
)
def _conv1_kernel(dis_hbm, h0_hbm, row_hbm, col_hbm, ew_hbm,
                  scat_hbm, coef_hbm,
                  dis_v, row_v, col_v, coef_v, gbuf, acc_sh, sem):
    c = lax.axis_index("c")
    s = lax.axis_index("s")
    w = s * 2 + c
    base = s * RPS

    def zg(i, _):
        for f in range(DIN // 16):
            gbuf[i, pl.ds(f * 16, 16)] = jnp.zeros((16,), jnp.float32)
        return 0
    lax.fori_loop(0, CHUNK, zg, 0)
    for r in range(4):
        pltpu.sync_copy(gbuf, acc_sh.at[pl.ds(base + r * CHUNK, CHUNK)])
    pltpu.sync_copy(gbuf.at[pl.ds(0, RPS - 4 * CHUNK)],
                    acc_sh.at[pl.ds(base + 4 * CHUNK, RPS - 4 * CHUNK)])
    pltpu.sync_copy(dis_hbm, dis_v)
    plsc.subcore_barrier()

    def chunk(j, _):
        pltpu.sync_copy(row_hbm.at[w, j], row_v)
        pltpu.sync_copy(col_hbm.at[w, j], col_v)
        pltpu.sync_copy(ew_hbm.at[w, j], coef_v)

        def grp(k, _):
            wv = _clean(coef_v[pl.ds(k * 16, 16)])
            dr = plsc.load_gather(dis_v, [row_v[pl.ds(k * 16, 16)]])
            dc = plsc.load_gather(dis_v, [col_v[pl.ds(k * 16, 16)]])
            coef_v[pl.ds(k * 16, 16)] = wv * dr * dc
            return 0
        lax.fori_loop(0, CHUNK // 16, grp, 0)
        pltpu.sync_copy(coef_v, coef_hbm.at[w, j])
        pltpu.async_copy(h0_hbm.at[row_v], gbuf, sem).wait()

        def scale(e, _):
            cf = plsc.load_gather(coef_v, [jnp.full((16,), e, jnp.int32)])
            for f in range(DIN // 16):
                gbuf[e, pl.ds(f * 16, 16)] = gbuf[e, pl.ds(f * 16, 16)] * cf
            return 0
        lax.fori_loop(0, CHUNK, scale, 0)
        pltpu.sync_copy(gbuf, acc_sh.at[col_v], add=True)
        return 0
    lax.fori_loop(0, NCH, chunk, 0)
    plsc.subcore_barrier()
    pltpu.sync_copy(acc_sh.at[pl.ds(base, RPS)],
                    scat_hbm.at[c, pl.ds(base, RPS)])


@functools.partial(
    pl.kernel,
    out_type=jax.ShapeDtypeStruct((2, N, DHF), jnp.float32),
    mesh=_MESH,
    compiler_params=_SCP,
    scratch_types=[
        pltpu.VMEM((CHUNK,), jnp.int32),
        pltpu.VMEM((CHUNK,), jnp.int32),
        pltpu.VMEM((CHUNK,), jnp.float32),
        pltpu.VMEM((CHUNK, DHF), jnp.float32),
        pltpu.VMEM_SHARED((N, DHF), jnp.float32),
        pltpu.SemaphoreType.DMA,
    ],
)
def _conv2_kernel(h1_hbm, row_hbm, col_hbm, coef_hbm, scat_hbm,
                  row_v, col_v, coef_v, gbuf, acc_sh, sem):
    c = lax.axis_index("c")
    s = lax.axis_index("s")
    base = s * RPS

    def zg(i, _):
        for f in range(DHF // 16):
            gbuf[i, pl.ds(f * 16, 16)] = jnp.zeros((16,), jnp.float32)
        return 0
    lax.fori_loop(0, CHUNK, zg, 0)
    for r in range(4):
        pltpu.sync_copy(gbuf, acc_sh.at[pl.ds(base + r * CHUNK, CHUNK)])
    pltpu.sync_copy(gbuf.at[pl.ds(0, RPS - 4 * CHUNK)],
                    acc_sh.at[pl.ds(base + 4 * CHUNK, RPS - 4 * CHUNK)])
    plsc.subcore_barrier()

    for half in range(2):
        w = s * 2 + half

        def chunk(j, _):
            pltpu.sync_copy(row_hbm.at[w, j], row_v)

            def adj(k, _):
                row_v[pl.ds(k * 16, 16)] = row_v[pl.ds(k * 16, 16)] + c * N
                return 0
            lax.fori_loop(0, CHUNK // 16, adj, 0)
            pltpu.sync_copy(col_hbm.at[w, j], col_v)
            pltpu.sync_copy(coef_hbm.at[w, j], coef_v)
            pltpu.async_copy(h1_hbm.at[row_v], gbuf, sem).wait()

            def scale(e, _):
                cf = plsc.load_gather(coef_v, [jnp.full((16,), e, jnp.int32)])
                for f in range(DHF // 16):
                    gbuf[e, pl.ds(f * 16, 16)] = gbuf[e, pl.ds(f * 16, 16)] * cf
                return 0
            lax.fori_loop(0, CHUNK, scale, 0)
            pltpu.sync_copy(gbuf, acc_sh.at[col_v], add=True)
            return 0
        lax.fori_loop(0, NCH, chunk, 0)
    plsc.subcore_barrier()
    pltpu.sync_copy(acc_sh.at[pl.ds(base, RPS)],
                    scat_hbm.at[c, pl.ds(base, RPS)])


def _stage_b_body(dp_ref, x_ref, g0_ref, b0_ref, h0_ref, dis_ref):
    @pl.when(pl.program_id(0) == 0)
    def _():
        deg = dp_ref[0] + dp_ref[1]
        dis_ref[...] = jnp.where(deg > 0, lax.rsqrt(deg), 0.0)
    h0_ref[...] = x_ref[...] * (g0_ref[...] * _BN0) + b0_ref[...]


def _stage_d_body(sc_ref, w1_ref, b1_ref, g1_ref, bb1_ref, h1_ref):
    agg = sc_ref[0] + sc_ref[1]
    a1 = lax.dot_general(agg, w1_ref[...], (((1,), (0,)), ((), ())),
                         precision=_HI,
                         preferred_element_type=jnp.float32) + b1_ref[...]
    h = jnp.maximum(a1 * (g1_ref[...] * _BN0) + bb1_ref[...], 0.0)
    h1_ref[0] = h[:, :DHF]
    h1_ref[1] = h[:, DHF:]


def _stage_f_body(sc_ref, w2_ref, b2_ref, g2_ref, bb2_ref, batch_ref,
                  wc1_ref, bc1_ref, wc2_ref, bc2_ref, out_ref,
                  pool_acc, cnt_acc):
    i = pl.program_id(0)

    @pl.when(i == 0)
    def _():
        pool_acc[...] = jnp.zeros((NG, DH), jnp.float32)
        cnt_acc[...] = jnp.zeros((NG, 128), jnp.float32)

    agg = jnp.concatenate([sc_ref[0], sc_ref[1]], axis=1)
    a2 = lax.dot_general(agg, w2_ref[...], (((1,), (0,)), ((), ())),
                         precision=_HI,
                         preferred_element_type=jnp.float32) + b2_ref[...]
    h = jnp.maximum(a2 * (g2_ref[...] * _BN0) + bb2_ref[...], 0.0)
    oh = (batch_ref[...] == lax.broadcasted_iota(jnp.int32, (1, NG), 1))
    oh = oh.astype(jnp.float32)
    pool_acc[...] += lax.dot_general(oh, h, (((0,), (0,)), ((), ())),
                                     precision=_HI,
                                     preferred_element_type=jnp.float32)
    cnt_acc[...] += lax.dot_general(oh, jnp.ones((NBLK, 128), jnp.float32),
                                    (((0,), (0,)), ((), ())),
                                    precision=_HI,
                                    preferred_element_type=jnp.float32)

    @pl.when(i == GRID - 1)
    def _():
        cnt = jnp.maximum(cnt_acc[:, 0:1], 1.0)
        pooled = pool_acc[...] / cnt
        z = lax.dot_general(pooled, wc1_ref[...], (((1,), (0,)), ((), ())),
                            precision=_HI,
                            preferred_element_type=jnp.float32) + bc1_ref[...]
        z = jnp.maximum(z, 0.0)
        out_ref[...] = lax.dot_general(z, wc2_ref[...],
                                       (((1,), (0,)), ((), ())),
                                       precision=_HI,
                                       preferred_element_type=jnp.float32
                                       ) + bc2_ref[...]


def _rows(shape):
    return pl.BlockSpec(shape, lambda i: (i,) + (0,) * (len(shape) - 1))


def _const(shape):
    return pl.BlockSpec(shape, lambda i: (0,) * len(shape))


_stage_b = pl.pallas_call(
    _stage_b_body,
    grid=(GRID,),
    in_specs=[_const((2, NROW, 128)), _rows((NBLK, DIN)),
              _const((1, DIN)), _const((1, DIN))],
    out_specs=[_rows((NBLK, DIN)), _const((NROW, 128))],
    out_shape=[jax.ShapeDtypeStruct((N, DIN), jnp.float32),
               jax.ShapeDtypeStruct((NROW, 128), jnp.float32)],
)

_stage_d = pl.pallas_call(
    _stage_d_body,
    grid=(GRID,),
    in_specs=[pl.BlockSpec((2, NBLK, DIN), lambda i: (0, i, 0)),
              _const((DIN, DH)), _const((1, DH)),
              _const((1, DH)), _const((1, DH))],
    out_specs=pl.BlockSpec((2, NBLK, DHF), lambda i: (0, i, 0)),
    out_shape=jax.ShapeDtypeStruct((2, N, DHF), jnp.float32),
)

_stage_f = pl.pallas_call(
    _stage_f_body,
    grid=(GRID,),
    in_specs=[pl.BlockSpec((2, NBLK, DHF), lambda i: (0, i, 0)),
              _const((DH, DH)), _const((1, DH)),
              _const((1, DH)), _const((1, DH)),
              _rows((NBLK, 1)),
              _const((DH, DH)), _const((1, DH)),
              _const((DH, 2)), _const((1, 2))],
    out_specs=_const((NG, 2)),
    out_shape=jax.ShapeDtypeStruct((NG, 2), jnp.float32),
    scratch_shapes=[pltpu.VMEM((NG, DH), jnp.float32),
                    pltpu.VMEM((NG, 128), jnp.float32)],
)


def kernel(x, edge_index, batch, edge_attr, bn0_g, bn0_b, W1, b1, bn1_g,
           bn1_b, W2, b2, bn2_g, bn2_b, Wc1, bc1, Wc2, bc2):
    sl = jnp.arange(N, dtype=jnp.int32)
    row = jnp.concatenate([edge_index[0].astype(jnp.int32), sl])
    col = jnp.concatenate([edge_index[1].astype(jnp.int32), sl])
    ew = jnp.concatenate([edge_attr, jnp.ones((N,), jnp.float32)])
    pad = E2P - row.shape[0]
    row = jnp.pad(row, (0, pad)).reshape(32, NCH, CHUNK)
    col = jnp.pad(col, (0, pad)).reshape(32, NCH, CHUNK)
    ew = jnp.pad(ew, (0, pad)).reshape(32, NCH, CHUNK)
    batch2 = batch.astype(jnp.int32).reshape(N, 1)
    r1 = lambda a: a.reshape(1, -1)

    deg_parts = _deg_kernel(col, ew)
    h0, dis = _stage_b(deg_parts, x, r1(bn0_g), r1(bn0_b))
    scat1, coef = _conv1_kernel(dis.reshape(NPAD), h0, row, col, ew)
    h1 = _stage_d(scat1, W1, r1(b1), r1(bn1_g), r1(bn1_b))
    scat2 = _conv2_kernel(h1.reshape(2 * N, DHF), row, col, coef)
    return _stage_f(scat2, W2, r1(b2), r1(bn2_g), r1(bn2_b), batch2,
                    Wc1, r1(bc1), Wc2, r1(bc2))


# speedup vs baseline: 8.4925x; 8.4925x over previous
import functools

import numpy as np
import jax
import jax.numpy as jnp
from jax import lax
from jax.experimental import pallas as pl
from jax.experimental.pallas import tpu as pltpu
from jax.experimental.pallas import tpu_sc as plsc

N = 10000
DIN = 128
DH = 256
DHF = 128
NG = 64
EPS = 1e-5
NROW = 80
NPAD = NROW * 128
CHUNK = 128
NCH = 81
E2P = 32 * NCH * CHUNK
RPS = 632
NBLK = 1000
GRID = N // NBLK

_MESH = plsc.VectorSubcoreMesh(core_axis_name="c", subcore_axis_name="s")
_SCP = pltpu.CompilerParams(needs_layout_passes=False)
_HI = lax.Precision.HIGHEST
_BN0 = np.float32(1.0 / np.sqrt(1.0 + EPS))


def _clean(w):
    w = jnp.where(w != w, 0.0, w)
    return jnp.where(jnp.abs(w) == jnp.inf, 0.0, w)


@functools.partial(
    pl.kernel,
    out_type=jax.ShapeDtypeStruct((2, NROW, 128), jnp.float32),
    mesh=_MESH,
    compiler_params=_SCP,
    scratch_types=[
        pltpu.VMEM((CHUNK,), jnp.int32),
        pltpu.VMEM((CHUNK,), jnp.float32),
        pltpu.VMEM((NROW, 128), jnp.float32),
        pltpu.VMEM((NROW,), jnp.int32),
        pltpu.VMEM_SHARED((NROW, 128), jnp.float32),
    ],
)
def _deg_kernel(col_hbm, ew_hbm, out_hbm, col_v, ew_v, acc_v, idx_v, acc_sh):
    c = lax.axis_index("c")
    s = lax.axis_index("s")
    w = s * 2 + c

    def zrow(i, _):
        for f in range(8):
            acc_v[i, pl.ds(f * 16, 16)] = jnp.zeros((16,), jnp.float32)
        return 0
    lax.fori_loop(0, NROW, zrow, 0)

    @pl.when(s < 10)
    def _():
        pltpu.sync_copy(acc_v.at[pl.ds(0, 8)], acc_sh.at[pl.ds(s * 8, 8)])

    def mkidx(k, _):
        idx_v[pl.ds(k * 16, 16)] = lax.iota(jnp.int32, 16) + k * 16
        return 0
    lax.fori_loop(0, NROW // 16, mkidx, 0)
    plsc.subcore_barrier()

    def chunk(j, _):
        pltpu.sync_copy(col_hbm.at[w, j], col_v)
        pltpu.sync_copy(ew_hbm.at[w, j], ew_v)

        def grp(k, _):
            cv = col_v[pl.ds(k * 16, 16)]
            wv = _clean(ew_v[pl.ds(k * 16, 16)])
            i0 = lax.shift_right_logical(cv, 7)
            i1 = lax.bitwise_and(cv, 127)
            plsc.addupdate_scatter(acc_v, [i0, i1], wv)
            return 0
        lax.fori_loop(0, CHUNK // 16, grp, 0)
        return 0
    lax.fori_loop(0, NCH, chunk, 0)

    pltpu.sync_copy(acc_v, acc_sh.at[idx_v], add=True)
    plsc.subcore_barrier()

    @pl.when(s < 10)
    def _():
        pltpu.sync_copy(acc_sh.at[pl.ds(s * 8, 8)],
                        out_hbm.at[c, pl.ds(s * 8, 8)])


@functools.partial(
    pl.kernel,
    out_type=(
        jax.ShapeDtypeStruct((2, N, DIN), jnp.float32),
        jax.ShapeDtypeStruct((32, NCH, CHUNK), jnp.float32),
    ),
    mesh=_MESH,
    compiler_params=_SCP,
    scratch_types=[
        pltpu.VMEM((NPAD,), jnp.float32),
        pltpu.VMEM((CHUNK,), jnp.int32),
        pltpu.VMEM((CHUNK,), jnp.int32),
        pltpu.VMEM((CHUNK,), jnp.float32),
        pltpu.VMEM((CHUNK, DIN), jnp.float32),
        pltpu.VMEM_SHARED((N, DIN), jnp.float32),
        pltpu.SemaphoreType.DMA,
    ],
)
def _conv1_kernel(dis_hbm, h0_hbm, row_hbm, col_hbm, ew_hbm,
                  scat_hbm, coef_hbm,
                  dis_v, row_v, col_v, coef_v, gbuf, acc_sh, sem):
    c = lax.axis_index("c")
    s = lax.axis_index("s")
    w = s * 2 + c
    base = s * RPS

    def zg(i, _):
        for f in range(DIN // 16):
            gbuf[i, pl.ds(f * 16, 16)] = jnp.zeros((16,), jnp.float32)
        return 0
    lax.fori_loop(0, CHUNK, zg, 0)
    for r in range(4):
        pltpu.sync_copy(gbuf, acc_sh.at[pl.ds(base + r * CHUNK, CHUNK)])

    @pl.when(s < 15)
    def _():
        pltpu.sync_copy(gbuf.at[pl.ds(0, 120)],
                        acc_sh.at[pl.ds(base + 512, 120)])

    @pl.when(s == 15)
    def _():
        pltpu.sync_copy(gbuf.at[pl.ds(0, 8)],
                        acc_sh.at[pl.ds(base + 512, 8)])
    pltpu.sync_copy(dis_hbm, dis_v)
    plsc.subcore_barrier()

    def chunk(j, _):
        pltpu.sync_copy(row_hbm.at[w, j], row_v)
        pltpu.sync_copy(col_hbm.at[w, j], col_v)
        pltpu.sync_copy(ew_hbm.at[w, j], coef_v)

        def grp(k, _):
            wv = _clean(coef_v[pl.ds(k * 16, 16)])
            dr = plsc.load_gather(dis_v, [row_v[pl.ds(k * 16, 16)]])
            dc = plsc.load_gather(dis_v, [col_v[pl.ds(k * 16, 16)]])
            coef_v[pl.ds(k * 16, 16)] = wv * dr * dc
            return 0
        lax.fori_loop(0, CHUNK // 16, grp, 0)
        pltpu.sync_copy(coef_v, coef_hbm.at[w, j])
        pltpu.async_copy(h0_hbm.at[row_v], gbuf, sem).wait()

        def scale(e, _):
            cf = plsc.load_gather(coef_v, [jnp.full((16,), e, jnp.int32)])
            for f in range(DIN // 16):
                gbuf[e, pl.ds(f * 16, 16)] = gbuf[e, pl.ds(f * 16, 16)] * cf
            return 0
        lax.fori_loop(0, CHUNK, scale, 0)
        pltpu.sync_copy(gbuf, acc_sh.at[col_v], add=True)
        return 0
    lax.fori_loop(0, NCH, chunk, 0)
    plsc.subcore_barrier()

    @pl.when(s < 15)
    def _():
        pltpu.sync_copy(acc_sh.at[pl.ds(base, 632)],
                        scat_hbm.at[c, pl.ds(base, 632)])

    @pl.when(s == 15)
    def _():
        pltpu.sync_copy(acc_sh.at[pl.ds(base, 520)],
                        scat_hbm.at[c, pl.ds(base, 520)])


@functools.partial(
    pl.kernel,
    out_type=jax.ShapeDtypeStruct((2, N, DHF), jnp.float32),
    mesh=_MESH,
    compiler_params=_SCP,
    scratch_types=[
        pltpu.VMEM((CHUNK,), jnp.int32),
        pltpu.VMEM((CHUNK,), jnp.int32),
        pltpu.VMEM((CHUNK,), jnp.float32),
        pltpu.VMEM((CHUNK, DHF), jnp.float32),
        pltpu.VMEM_SHARED((N, DHF), jnp.float32),
        pltpu.SemaphoreType.DMA,
    ],
)
def _conv2_kernel(h1_hbm, row_hbm, col_hbm, coef_hbm, scat_hbm,
                  row_v, col_v, coef_v, gbuf, acc_sh, sem):
    c = lax.axis_index("c")
    s = lax.axis_index("s")
    base = s * RPS

    def zg(i, _):
        for f in range(DHF // 16):
            gbuf[i, pl.ds(f * 16, 16)] = jnp.zeros((16,), jnp.float32)
        return 0
    lax.fori_loop(0, CHUNK, zg, 0)
    for r in range(4):
        pltpu.sync_copy(gbuf, acc_sh.at[pl.ds(base + r * CHUNK, CHUNK)])

    @pl.when(s < 15)
    def _():
        pltpu.sync_copy(gbuf.at[pl.ds(0, 120)],
                        acc_sh.at[pl.ds(base + 512, 120)])

    @pl.when(s == 15)
    def _():
        pltpu.sync_copy(gbuf.at[pl.ds(0, 8)],
                        acc_sh.at[pl.ds(base + 512, 8)])
    plsc.subcore_barrier()

    for half in range(2):
        w = s * 2 + half

        def chunk(j, _):
            pltpu.sync_copy(row_hbm.at[w, j], row_v)

            def adj(k, _):
                row_v[pl.ds(k * 16, 16)] = row_v[pl.ds(k * 16, 16)] + c * N
                return 0
            lax.fori_loop(0, CHUNK // 16, adj, 0)
            pltpu.sync_copy(col_hbm.at[w, j], col_v)
            pltpu.sync_copy(coef_hbm.at[w, j], coef_v)
            pltpu.async_copy(h1_hbm.at[row_v], gbuf, sem).wait()

            def scale(e, _):
                cf = plsc.load_gather(coef_v, [jnp.full((16,), e, jnp.int32)])
                for f in range(DHF // 16):
                    gbuf[e, pl.ds(f * 16, 16)] = gbuf[e, pl.ds(f * 16, 16)] * cf
                return 0
            lax.fori_loop(0, CHUNK, scale, 0)
            pltpu.sync_copy(gbuf, acc_sh.at[col_v], add=True)
            return 0
        lax.fori_loop(0, NCH, chunk, 0)
    plsc.subcore_barrier()

    @pl.when(s < 15)
    def _():
        pltpu.sync_copy(acc_sh.at[pl.ds(base, 632)],
                        scat_hbm.at[c, pl.ds(base, 632)])

    @pl.when(s == 15)
    def _():
        pltpu.sync_copy(acc_sh.at[pl.ds(base, 520)],
                        scat_hbm.at[c, pl.ds(base, 520)])


def _stage_b_body(dp_ref, x_ref, g0_ref, b0_ref, h0_ref, dis_ref):
    @pl.when(pl.program_id(0) == 0)
    def _():
        deg = dp_ref[0] + dp_ref[1]
        dis_ref[...] = jnp.where(deg > 0, lax.rsqrt(deg), 0.0)
    h0_ref[...] = x_ref[...] * (g0_ref[...] * _BN0) + b0_ref[...]


def _stage_d_body(sc_ref, w1_ref, b1_ref, g1_ref, bb1_ref, h1_ref):
    agg = sc_ref[0] + sc_ref[1]
    a1 = lax.dot_general(agg, w1_ref[...], (((1,), (0,)), ((), ())),
                         precision=_HI,
                         preferred_element_type=jnp.float32) + b1_ref[...]
    h = jnp.maximum(a1 * (g1_ref[...] * _BN0) + bb1_ref[...], 0.0)
    h1_ref[0] = h[:, :DHF]
    h1_ref[1] = h[:, DHF:]


def _stage_f_body(sc_ref, w2_ref, b2_ref, g2_ref, bb2_ref, batch_ref,
                  wc1_ref, bc1_ref, wc2_ref, bc2_ref, out_ref,
                  pool_acc, cnt_acc):
    i = pl.program_id(0)

    @pl.when(i == 0)
    def _():
        pool_acc[...] = jnp.zeros((NG, DH), jnp.float32)
        cnt_acc[...] = jnp.zeros((NG, 128), jnp.float32)

    agg = jnp.concatenate([sc_ref[0], sc_ref[1]], axis=1)
    a2 = lax.dot_general(agg, w2_ref[...], (((1,), (0,)), ((), ())),
                         precision=_HI,
                         preferred_element_type=jnp.float32) + b2_ref[...]
    h = jnp.maximum(a2 * (g2_ref[...] * _BN0) + bb2_ref[...], 0.0)
    oh = (batch_ref[...] == lax.broadcasted_iota(jnp.int32, (1, NG), 1))
    oh = oh.astype(jnp.float32)
    pool_acc[...] += lax.dot_general(oh, h, (((0,), (0,)), ((), ())),
                                     precision=_HI,
                                     preferred_element_type=jnp.float32)
    cnt_acc[...] += lax.dot_general(oh, jnp.ones((NBLK, 128), jnp.float32),
                                    (((0,), (0,)), ((), ())),
                                    precision=_HI,
                                    preferred_element_type=jnp.float32)

    @pl.when(i == GRID - 1)
    def _():
        cnt = jnp.maximum(cnt_acc[:, 0:1], 1.0)
        pooled = pool_acc[...] / cnt
        z = lax.dot_general(pooled, wc1_ref[...], (((1,), (0,)), ((), ())),
                            precision=_HI,
                            preferred_element_type=jnp.float32) + bc1_ref[...]
        z = jnp.maximum(z, 0.0)
        out_ref[...] = lax.dot_general(z, wc2_ref[...],
                                       (((1,), (0,)), ((), ())),
                                       precision=_HI,
                                       preferred_element_type=jnp.float32
                                       ) + bc2_ref[...]


def _rows(shape):
    return pl.BlockSpec(shape, lambda i: (i,) + (0,) * (len(shape) - 1))


def _const(shape):
    return pl.BlockSpec(shape, lambda i: (0,) * len(shape))


_stage_b = pl.pallas_call(
    _stage_b_body,
    grid=(GRID,),
    in_specs=[_const((2, NROW, 128)), _rows((NBLK, DIN)),
              _const((1, DIN)), _const((1, DIN))],
    out_specs=[_rows((NBLK, DIN)), _const((NROW, 128))],
    out_shape=[jax.ShapeDtypeStruct((N, DIN), jnp.float32),
               jax.ShapeDtypeStruct((NROW, 128), jnp.float32)],
)

_stage_d = pl.pallas_call(
    _stage_d_body,
    grid=(GRID,),
    in_specs=[pl.BlockSpec((2, NBLK, DIN), lambda i: (0, i, 0)),
              _const((DIN, DH)), _const((1, DH)),
              _const((1, DH)), _const((1, DH))],
    out_specs=pl.BlockSpec((2, NBLK, DHF), lambda i: (0, i, 0)),
    out_shape=jax.ShapeDtypeStruct((2, N, DHF), jnp.float32),
)

_stage_f = pl.pallas_call(
    _stage_f_body,
    grid=(GRID,),
    in_specs=[pl.BlockSpec((2, NBLK, DHF), lambda i: (0, i, 0)),
              _const((DH, DH)), _const((1, DH)),
              _const((1, DH)), _const((1, DH)),
              _rows((NBLK, 1)),
              _const((DH, DH)), _const((1, DH)),
              _const((DH, 2)), _const((1, 2))],
    out_specs=_const((NG, 2)),
    out_shape=jax.ShapeDtypeStruct((NG, 2), jnp.float32),
    scratch_shapes=[pltpu.VMEM((NG, DH), jnp.float32),
                    pltpu.VMEM((NG, 128), jnp.float32)],
)


def kernel(x, edge_index, batch, edge_attr, bn0_g, bn0_b, W1, b1, bn1_g,
           bn1_b, W2, b2, bn2_g, bn2_b, Wc1, bc1, Wc2, bc2):
    sl = jnp.arange(N, dtype=jnp.int32)
    row = jnp.concatenate([edge_index[0].astype(jnp.int32), sl])
    col = jnp.concatenate([edge_index[1].astype(jnp.int32), sl])
    ew = jnp.concatenate([edge_attr, jnp.ones((N,), jnp.float32)])
    pad = E2P - row.shape[0]
    row = jnp.pad(row, (0, pad)).reshape(32, NCH, CHUNK)
    col = jnp.pad(col, (0, pad)).reshape(32, NCH, CHUNK)
    ew = jnp.pad(ew, (0, pad)).reshape(32, NCH, CHUNK)
    batch2 = batch.astype(jnp.int32).reshape(N, 1)
    r1 = lambda a: a.reshape(1, -1)

    deg_parts = _deg_kernel(col, ew)
    h0, dis = _stage_b(deg_parts, x, r1(bn0_g), r1(bn0_b))
    scat1, coef = _conv1_kernel(dis.reshape(NPAD), h0, row, col, ew)
    h1 = _stage_d(scat1, W1, r1(b1), r1(bn1_g), r1(bn1_b))
    scat2 = _conv2_kernel(h1.reshape(2 * N, DHF), row, col, coef)
    return _stage_f(scat2, W2, r1(b2), r1(bn2_g), r1(bn2_b), batch2,
                    Wc1, r1(bc1), Wc2, r1(bc2))


# speedup vs baseline: 16.7705x; 1.9747x over previous
import functools

import numpy as np
import jax
import jax.numpy as jnp
from jax import lax
from jax.experimental import pallas as pl
from jax.experimental.pallas import tpu as pltpu
from jax.experimental.pallas import tpu_sc as plsc

N = 10000
DIN = 128
DH = 256
DHF = 128
NG = 64
EPS = 1e-5
NROW = 80
NPAD = NROW * 128
CHUNK = 64
NCH = 162
E2P = 32 * NCH * CHUNK
RPS = 632
NB = 3
NBLK = 1000
GRID = N // NBLK

_MESH = plsc.VectorSubcoreMesh(core_axis_name="c", subcore_axis_name="s")
_SCP = pltpu.CompilerParams(needs_layout_passes=False)
_HI = lax.Precision.HIGHEST
_BN0 = np.float32(1.0 / np.sqrt(1.0 + EPS))


def _clean(w):
    w = jnp.where(w != w, 0.0, w)
    return jnp.where(jnp.abs(w) == jnp.inf, 0.0, w)


def _splat(v):
    return jnp.full((16,), v, jnp.int32)


def _unpack_row(p):
    return lax.bitwise_and(p, 16383)


def _unpack_col(p):
    return lax.shift_right_logical(p, 14)


@functools.partial(
    pl.kernel,
    out_type=jax.ShapeDtypeStruct((2, NROW, 128), jnp.float32),
    mesh=_MESH,
    compiler_params=_SCP,
    scratch_types=[
        pltpu.VMEM((NCH, CHUNK), jnp.int32),
        pltpu.VMEM((NCH, CHUNK), jnp.float32),
        pltpu.VMEM((NROW, 128), jnp.float32),
        pltpu.VMEM((NROW,), jnp.int32),
        pltpu.VMEM_SHARED((NROW, 128), jnp.float32),
    ],
)
def _deg_kernel(pk_hbm, ew_hbm, out_hbm, pk_b, ew_b, acc_v, idx_v, acc_sh):
    c = lax.axis_index("c")
    s = lax.axis_index("s")
    w = s * 2 + c

    def zrow(i, _):
        for f in range(8):
            acc_v[i, pl.ds(f * 16, 16)] = jnp.zeros((16,), jnp.float32)
        return 0
    lax.fori_loop(0, NROW, zrow, 0)

    @pl.when(s < 10)
    def _():
        pltpu.sync_copy(acc_v.at[pl.ds(0, 8)], acc_sh.at[pl.ds(s * 8, 8)])

    def mkidx(k, _):
        idx_v[pl.ds(k * 16, 16)] = lax.iota(jnp.int32, 16) + k * 16
        return 0
    lax.fori_loop(0, NROW // 16, mkidx, 0)
    pltpu.sync_copy(pk_hbm.at[w], pk_b)
    pltpu.sync_copy(ew_hbm.at[w], ew_b)
    plsc.subcore_barrier()

    def hist(t, _):
        q = lax.shift_right_logical(t, 2)
        k = lax.bitwise_and(t, 3)
        dsk = pl.ds(k * 16, 16)
        cv = _unpack_col(pk_b[q, dsk])
        wv = _clean(ew_b[q, dsk])
        i0 = lax.shift_right_logical(cv, 7)
        i1 = lax.bitwise_and(cv, 127)
        plsc.addupdate_scatter(acc_v, [i0, i1], wv)
        return 0
    lax.fori_loop(0, NCH * (CHUNK // 16), hist, 0)

    pltpu.sync_copy(acc_v, acc_sh.at[idx_v], add=True)
    plsc.subcore_barrier()

    @pl.when(s < 10)
    def _():
        pltpu.sync_copy(acc_sh.at[pl.ds(s * 8, 8)],
                        out_hbm.at[c, pl.ds(s * 8, 8)])


@functools.partial(
    pl.kernel,
    out_type=jax.ShapeDtypeStruct((32, NCH, CHUNK), jnp.float32),
    mesh=_MESH,
    compiler_params=_SCP,
    scratch_types=[
        pltpu.VMEM((NPAD,), jnp.float32),
        pltpu.VMEM((NCH, CHUNK), jnp.int32),
        pltpu.VMEM((NCH, CHUNK), jnp.float32),
    ],
)
def _coef_kernel(dis_hbm, pk_hbm, ew_hbm, coef_hbm, dis_v, pk_b, coef_b):
    c = lax.axis_index("c")
    s = lax.axis_index("s")
    w = s * 2 + c
    pltpu.sync_copy(dis_hbm, dis_v)
    pltpu.sync_copy(pk_hbm.at[w], pk_b)
    pltpu.sync_copy(ew_hbm.at[w], coef_b)

    def mkc(t, _):
        q = lax.shift_right_logical(t, 2)
        k = lax.bitwise_and(t, 3)
        dsk = pl.ds(k * 16, 16)
        p = pk_b[q, dsk]
        wv = _clean(coef_b[q, dsk])
        dr = plsc.load_gather(dis_v, [_unpack_row(p)])
        dc = plsc.load_gather(dis_v, [_unpack_col(p)])
        coef_b[q, dsk] = wv * dr * dc
        return 0
    lax.fori_loop(0, NCH * (CHUNK // 16), mkc, 0)
    pltpu.sync_copy(coef_b, coef_hbm.at[w])


def _conv_body(two_passes):
    def body(tbl_hbm, pk_hbm, coef_hbm, scat_hbm,
             pk_b, coef_b, rows_b, cols_b, gbuf, acc_sh,
             gs0, gs1, gs2, ss0, ss1, ss2):
        c = lax.axis_index("c")
        s = lax.axis_index("s")
        gbufs = [gbuf.at[b] for b in range(NB)]
        gsems = [gs0, gs1, gs2]
        ssems = [ss0, ss1, ss2]
        base = s * RPS

        def zrow(i, _):
            for f in range(DHF // 16):
                gbuf[0, i, pl.ds(f * 16, 16)] = jnp.zeros((16,), jnp.float32)
            return 0
        lax.fori_loop(0, CHUNK, zrow, 0)
        for r in range(RPS // CHUNK):
            pltpu.sync_copy(gbufs[0],
                            acc_sh.at[pl.ds(base + r * CHUNK, CHUNK)])

        @pl.when(s < 15)
        def _():
            pltpu.sync_copy(gbufs[0].at[pl.ds(0, RPS - (RPS // CHUNK) * CHUNK)],
                            acc_sh.at[pl.ds(base + (RPS // CHUNK) * CHUNK,
                                            RPS - (RPS // CHUNK) * CHUNK)])

        @pl.when(s == 15)
        def _():
            pltpu.sync_copy(gbufs[0].at[pl.ds(0, 8)],
                            acc_sh.at[pl.ds(base + (RPS // CHUNK) * CHUNK, 8)])
        plsc.subcore_barrier()

        if two_passes:
            passes = [(s * 2, c * N), (s * 2 + 1, c * N)]
        else:
            passes = [(s * 2 + c, jnp.int32(0))]

        for w, off in passes:
            pltpu.sync_copy(pk_hbm.at[w], pk_b)
            pltpu.sync_copy(coef_hbm.at[w], coef_b)

            def unpack(q, slot):
                def u(k, _):
                    dsk = pl.ds(k * 16, 16)
                    p = pk_b[pl.ds(q * CHUNK + k * 16, 16)]
                    rows_b[slot, dsk] = _unpack_row(p) + off
                    cols_b[slot, dsk] = _unpack_col(p)
                    return 0
                lax.fori_loop(0, CHUNK // 16, u, 0)

            def gath(q, b):
                pltpu.async_copy(tbl_hbm.at[rows_b.at[b]], gbufs[b], gsems[b])

            def wait_g(b):
                pltpu.make_async_copy(tbl_hbm.at[rows_b.at[b]], gbufs[b],
                                      gsems[b]).wait()

            def scat(b):
                pltpu.async_copy(gbufs[b], acc_sh.at[cols_b.at[b]], ssems[b],
                                 add=True)

            def wait_s(b):
                pltpu.make_async_copy(gbufs[b], acc_sh.at[cols_b.at[b]],
                                      ssems[b]).wait()

            def scale(q, b):
                gb = gbufs[b]

                def sc_e(e, _):
                    cf = plsc.load_gather(coef_b, [_splat(q * CHUNK + e)])
                    for f in range(DHF // 16):
                        gb[e, pl.ds(f * 16, 16)] = (
                            gb[e, pl.ds(f * 16, 16)] * cf)
                    return 0
                lax.fori_loop(0, CHUNK, sc_e, 0)

            for b in range(NB):
                unpack(b, b)
                gath(b, b)
            wait_g(0)
            scale(0, 0)
            scat(0)
            for b in range(1, NB):
                wait_g(b)
                scale(b, b)
                wait_s(b - 1)
                unpack(b + NB - 1, (b + NB - 1) % NB)
                gath(b + NB - 1, (b + NB - 1) % NB)
                scat(b)

            def blk(g, _):
                for b in range(NB):
                    q = g * NB + b
                    pv = (b - 1) % NB
                    wait_g(b)
                    scale(q, b)
                    wait_s(pv)
                    qn = q + NB - 1

                    @pl.when(qn < NCH)
                    def _():
                        unpack(qn, pv)
                        gath(qn, pv)
                    scat(b)
                return 0
            lax.fori_loop(1, NCH // NB, blk, 0)
            wait_s((NCH - 1) % NB)

        plsc.subcore_barrier()

        @pl.when(s < 15)
        def _():
            pltpu.sync_copy(acc_sh.at[pl.ds(base, 632)],
                            scat_hbm.at[c, pl.ds(base, 632)])

        @pl.when(s == 15)
        def _():
            pltpu.sync_copy(acc_sh.at[pl.ds(base, 520)],
                            scat_hbm.at[c, pl.ds(base, 520)])
    return body


_CONV_SCRATCH = [
    pltpu.VMEM((NCH * CHUNK,), jnp.int32),
    pltpu.VMEM((NCH * CHUNK,), jnp.float32),
    pltpu.VMEM((NB, CHUNK), jnp.int32),
    pltpu.VMEM((NB, CHUNK), jnp.int32),
    pltpu.VMEM((NB, CHUNK, DHF), jnp.float32),
    pltpu.VMEM_SHARED((N, DHF), jnp.float32),
    pltpu.SemaphoreType.DMA,
    pltpu.SemaphoreType.DMA,
    pltpu.SemaphoreType.DMA,
    pltpu.SemaphoreType.DMA,
    pltpu.SemaphoreType.DMA,
    pltpu.SemaphoreType.DMA,
]

_conv1_kernel = pl.kernel(
    _conv_body(False),
    out_type=jax.ShapeDtypeStruct((2, N, DIN), jnp.float32),
    mesh=_MESH,
    compiler_params=_SCP,
    scratch_types=_CONV_SCRATCH,
)

_conv2_kernel = pl.kernel(
    _conv_body(True),
    out_type=jax.ShapeDtypeStruct((2, N, DHF), jnp.float32),
    mesh=_MESH,
    compiler_params=_SCP,
    scratch_types=_CONV_SCRATCH,
)


def _stage_b_body(dp_ref, x_ref, g0_ref, b0_ref, h0_ref, dis_ref):
    @pl.when(pl.program_id(0) == 0)
    def _():
        deg = dp_ref[0] + dp_ref[1]
        dis_ref[...] = jnp.where(deg > 0, lax.rsqrt(deg), 0.0)
    h0_ref[...] = x_ref[...] * (g0_ref[...] * _BN0) + b0_ref[...]


def _stage_d_body(sc_ref, w1_ref, b1_ref, g1_ref, bb1_ref, h1_ref):
    agg = sc_ref[0] + sc_ref[1]
    a1 = lax.dot_general(agg, w1_ref[...], (((1,), (0,)), ((), ())),
                         precision=_HI,
                         preferred_element_type=jnp.float32) + b1_ref[...]
    h = jnp.maximum(a1 * (g1_ref[...] * _BN0) + bb1_ref[...], 0.0)
    h1_ref[0] = h[:, :DHF]
    h1_ref[1] = h[:, DHF:]


def _stage_f_body(sc_ref, w2_ref, b2_ref, g2_ref, bb2_ref, batch_ref,
                  wc1_ref, bc1_ref, wc2_ref, bc2_ref, out_ref,
                  pool_acc, cnt_acc):
    i = pl.program_id(0)

    @pl.when(i == 0)
    def _():
        pool_acc[...] = jnp.zeros((NG, DH), jnp.float32)
        cnt_acc[...] = jnp.zeros((NG, 128), jnp.float32)

    agg = jnp.concatenate([sc_ref[0], sc_ref[1]], axis=1)
    a2 = lax.dot_general(agg, w2_ref[...], (((1,), (0,)), ((), ())),
                         precision=_HI,
                         preferred_element_type=jnp.float32) + b2_ref[...]
    h = jnp.maximum(a2 * (g2_ref[...] * _BN0) + bb2_ref[...], 0.0)
    oh = (batch_ref[...] == lax.broadcasted_iota(jnp.int32, (1, NG), 1))
    oh = oh.astype(jnp.float32)
    pool_acc[...] += lax.dot_general(oh, h, (((0,), (0,)), ((), ())),
                                     precision=_HI,
                                     preferred_element_type=jnp.float32)
    cnt_acc[...] += lax.dot_general(oh, jnp.ones((NBLK, 128), jnp.float32),
                                    (((0,), (0,)), ((), ())),
                                    precision=_HI,
                                    preferred_element_type=jnp.float32)

    @pl.when(i == GRID - 1)
    def _():
        cnt = jnp.maximum(cnt_acc[:, 0:1], 1.0)
        pooled = pool_acc[...] / cnt
        z = lax.dot_general(pooled, wc1_ref[...], (((1,), (0,)), ((), ())),
                            precision=_HI,
                            preferred_element_type=jnp.float32) + bc1_ref[...]
        z = jnp.maximum(z, 0.0)
        out_ref[...] = lax.dot_general(z, wc2_ref[...],
                                       (((1,), (0,)), ((), ())),
                                       precision=_HI,
                                       preferred_element_type=jnp.float32
                                       ) + bc2_ref[...]


def _rows(shape):
    return pl.BlockSpec(shape, lambda i: (i,) + (0,) * (len(shape) - 1))


def _const(shape):
    return pl.BlockSpec(shape, lambda i: (0,) * len(shape))


_stage_b = pl.pallas_call(
    _stage_b_body,
    grid=(GRID,),
    in_specs=[_const((2, NROW, 128)), _rows((NBLK, DIN)),
              _const((1, DIN)), _const((1, DIN))],
    out_specs=[_rows((NBLK, DIN)), _const((NROW, 128))],
    out_shape=[jax.ShapeDtypeStruct((N, DIN), jnp.float32),
               jax.ShapeDtypeStruct((NROW, 128), jnp.float32)],
)

_stage_d = pl.pallas_call(
    _stage_d_body,
    grid=(GRID,),
    in_specs=[pl.BlockSpec((2, NBLK, DIN), lambda i: (0, i, 0)),
              _const((DIN, DH)), _const((1, DH)),
              _const((1, DH)), _const((1, DH))],
    out_specs=pl.BlockSpec((2, NBLK, DHF), lambda i: (0, i, 0)),
    out_shape=jax.ShapeDtypeStruct((2, N, DHF), jnp.float32),
)

_stage_f = pl.pallas_call(
    _stage_f_body,
    grid=(GRID,),
    in_specs=[pl.BlockSpec((2, NBLK, DHF), lambda i: (0, i, 0)),
              _const((DH, DH)), _const((1, DH)),
              _const((1, DH)), _const((1, DH)),
              _rows((NBLK, 1)),
              _const((DH, DH)), _const((1, DH)),
              _const((DH, 2)), _const((1, 2))],
    out_specs=_const((NG, 2)),
    out_shape=jax.ShapeDtypeStruct((NG, 2), jnp.float32),
    scratch_shapes=[pltpu.VMEM((NG, DH), jnp.float32),
                    pltpu.VMEM((NG, 128), jnp.float32)],
)


def kernel(x, edge_index, batch, edge_attr, bn0_g, bn0_b, W1, b1, bn1_g,
           bn1_b, W2, b2, bn2_g, bn2_b, Wc1, bc1, Wc2, bc2):
    sl = jnp.arange(N, dtype=jnp.int32)
    row = jnp.concatenate([edge_index[0].astype(jnp.int32), sl])
    col = jnp.concatenate([edge_index[1].astype(jnp.int32), sl])
    ew = jnp.concatenate([edge_attr, jnp.ones((N,), jnp.float32)])
    packed = jnp.bitwise_or(row, jnp.left_shift(col, 14))
    pad = E2P - packed.shape[0]
    packed = jnp.pad(packed, (0, pad)).reshape(32, NCH, CHUNK)
    ew = jnp.pad(ew, (0, pad)).reshape(32, NCH, CHUNK)
    batch2 = batch.astype(jnp.int32).reshape(N, 1)
    r1 = lambda a: a.reshape(1, -1)

    deg_parts = _deg_kernel(packed, ew)
    h0, dis = _stage_b(deg_parts, x, r1(bn0_g), r1(bn0_b))
    coef = _coef_kernel(dis.reshape(NPAD), packed, ew)
    packed2 = packed.reshape(32, NCH * CHUNK)
    scat1 = _conv1_kernel(h0, packed2, coef.reshape(32, NCH * CHUNK))
    h1 = _stage_d(scat1, W1, r1(b1), r1(bn1_g), r1(bn1_b))
    scat2 = _conv2_kernel(h1.reshape(2 * N, DHF), packed2,
                          coef.reshape(32, NCH * CHUNK))
    return _stage_f(scat2, W2, r1(b2), r1(bn2_g), r1(bn2_b), batch2,
                    Wc1, r1(bc1), Wc2, r1(bc2))


# speedup vs baseline: 16.9661x; 1.0117x over previous
import functools

import numpy as np
import jax
import jax.numpy as jnp
from jax import lax
from jax.experimental import pallas as pl
from jax.experimental.pallas import tpu as pltpu
from jax.experimental.pallas import tpu_sc as plsc

N = 10000
DIN = 128
DH = 256
DHF = 128
NG = 64
EPS = 1e-5
NROW = 80
NPAD = NROW * 128
CHUNK = 64
NCH = 162
E2P = 32 * NCH * CHUNK
RPS = 632
NB = 3
NBLK = 1000
GRID = N // NBLK

_MESH = plsc.VectorSubcoreMesh(core_axis_name="c", subcore_axis_name="s")
_SCP = pltpu.CompilerParams(needs_layout_passes=False)
_HI = lax.Precision.HIGHEST
_BN0 = np.float32(1.0 / np.sqrt(1.0 + EPS))


def _clean(w):
    w = jnp.where(w != w, 0.0, w)
    return jnp.where(jnp.abs(w) == jnp.inf, 0.0, w)


def _splat(v):
    return jnp.full((16,), v, jnp.int32)


def _unpack_row(p):
    return lax.bitwise_and(p, 16383)


def _unpack_col(p):
    return lax.shift_right_logical(p, 14)


@functools.partial(
    pl.kernel,
    out_type=jax.ShapeDtypeStruct((2, NROW, 128), jnp.float32),
    mesh=_MESH,
    compiler_params=_SCP,
    scratch_types=[
        pltpu.VMEM((NCH, CHUNK), jnp.int32),
        pltpu.VMEM((NCH, CHUNK), jnp.float32),
        pltpu.VMEM((NROW, 128), jnp.float32),
        pltpu.VMEM((NROW,), jnp.int32),
        pltpu.VMEM_SHARED((NROW, 128), jnp.float32),
    ],
)
def _deg_kernel(pk_hbm, ew_hbm, out_hbm, pk_b, ew_b, acc_v, idx_v, acc_sh):
    c = lax.axis_index("c")
    s = lax.axis_index("s")
    w = s * 2 + c

    def zrow(i, _):
        for f in range(8):
            acc_v[i, pl.ds(f * 16, 16)] = jnp.zeros((16,), jnp.float32)
        return 0
    lax.fori_loop(0, NROW, zrow, 0)

    @pl.when(s < 10)
    def _():
        pltpu.sync_copy(acc_v.at[pl.ds(0, 8)], acc_sh.at[pl.ds(s * 8, 8)])

    def mkidx(k, _):
        idx_v[pl.ds(k * 16, 16)] = lax.iota(jnp.int32, 16) + k * 16
        return 0
    lax.fori_loop(0, NROW // 16, mkidx, 0)
    pltpu.sync_copy(pk_hbm.at[w], pk_b)
    pltpu.sync_copy(ew_hbm.at[w], ew_b)
    plsc.subcore_barrier()

    def hist(t, _):
        q = lax.shift_right_logical(t, 2)
        k = lax.bitwise_and(t, 3)
        dsk = pl.ds(k * 16, 16)
        cv = _unpack_col(pk_b[q, dsk])
        wv = _clean(ew_b[q, dsk])
        i0 = lax.shift_right_logical(cv, 7)
        i1 = lax.bitwise_and(cv, 127)
        plsc.addupdate_scatter(acc_v, [i0, i1], wv)
        return 0
    lax.fori_loop(0, NCH * (CHUNK // 16), hist, 0)

    pltpu.sync_copy(acc_v, acc_sh.at[idx_v], add=True)
    plsc.subcore_barrier()

    @pl.when(s < 10)
    def _():
        pltpu.sync_copy(acc_sh.at[pl.ds(s * 8, 8)],
                        out_hbm.at[c, pl.ds(s * 8, 8)])


@functools.partial(
    pl.kernel,
    out_type=jax.ShapeDtypeStruct((32, NCH, CHUNK), jnp.float32),
    mesh=_MESH,
    compiler_params=_SCP,
    scratch_types=[
        pltpu.VMEM((NPAD,), jnp.float32),
        pltpu.VMEM((NCH, CHUNK), jnp.int32),
        pltpu.VMEM((NCH, CHUNK), jnp.float32),
    ],
)
def _coef_kernel(dis_hbm, pk_hbm, ew_hbm, coef_hbm, dis_v, pk_b, coef_b):
    c = lax.axis_index("c")
    s = lax.axis_index("s")
    w = s * 2 + c
    pltpu.sync_copy(dis_hbm, dis_v)
    pltpu.sync_copy(pk_hbm.at[w], pk_b)
    pltpu.sync_copy(ew_hbm.at[w], coef_b)

    def mkc(t, _):
        q = lax.shift_right_logical(t, 2)
        k = lax.bitwise_and(t, 3)
        dsk = pl.ds(k * 16, 16)
        p = pk_b[q, dsk]
        wv = _clean(coef_b[q, dsk])
        dr = plsc.load_gather(dis_v, [_unpack_row(p)])
        dc = plsc.load_gather(dis_v, [_unpack_col(p)])
        coef_b[q, dsk] = wv * dr * dc
        return 0
    lax.fori_loop(0, NCH * (CHUNK // 16), mkc, 0)
    pltpu.sync_copy(coef_b, coef_hbm.at[w])


def _conv_body(two_passes):
    def body(tbl_hbm, pk_hbm, coef_hbm, scat_hbm,
             pk_b, coef_b, rows_b, cols_b, gbuf, acc_sh,
             gs0, gs1, gs2, ss0, ss1, ss2):
        c = lax.axis_index("c")
        s = lax.axis_index("s")
        gbufs = [gbuf.at[b] for b in range(NB)]
        gsems = [gs0, gs1, gs2]
        ssems = [ss0, ss1, ss2]
        base = s * RPS

        def zrow(i, _):
            for f in range(DHF // 16):
                gbuf[0, i, pl.ds(f * 16, 16)] = jnp.zeros((16,), jnp.float32)
            return 0
        lax.fori_loop(0, CHUNK, zrow, 0)
        for r in range(RPS // CHUNK):
            pltpu.sync_copy(gbufs[0],
                            acc_sh.at[pl.ds(base + r * CHUNK, CHUNK)])

        @pl.when(s < 15)
        def _():
            pltpu.sync_copy(gbufs[0].at[pl.ds(0, RPS - (RPS // CHUNK) * CHUNK)],
                            acc_sh.at[pl.ds(base + (RPS // CHUNK) * CHUNK,
                                            RPS - (RPS // CHUNK) * CHUNK)])

        @pl.when(s == 15)
        def _():
            pltpu.sync_copy(gbufs[0].at[pl.ds(0, 8)],
                            acc_sh.at[pl.ds(base + (RPS // CHUNK) * CHUNK, 8)])
        plsc.subcore_barrier()

        if two_passes:
            passes = [(s * 2, c * N), (s * 2 + 1, c * N)]
        else:
            passes = [(s * 2 + c, jnp.int32(0))]

        for w, off in passes:
            pltpu.sync_copy(pk_hbm.at[w], pk_b)
            pltpu.sync_copy(coef_hbm.at[w], coef_b)

            def unpack(q, slot):
                def u(k, _):
                    dsk = pl.ds(k * 16, 16)
                    p = pk_b[pl.ds(q * CHUNK + k * 16, 16)]
                    rows_b[slot, dsk] = _unpack_row(p) + off
                    cols_b[slot, dsk] = _unpack_col(p)
                    return 0
                lax.fori_loop(0, CHUNK // 16, u, 0)

            def gath(q, b):
                pltpu.async_copy(tbl_hbm.at[rows_b.at[b]], gbufs[b], gsems[b])

            def wait_g(b):
                pltpu.make_async_copy(tbl_hbm.at[rows_b.at[b]], gbufs[b],
                                      gsems[b]).wait()

            def scat(b):
                pltpu.async_copy(gbufs[b], acc_sh.at[cols_b.at[b]], ssems[b],
                                 add=True)

            def wait_s(b):
                pltpu.make_async_copy(gbufs[b], acc_sh.at[cols_b.at[b]],
                                      ssems[b]).wait()

            def scale(q, b):
                gb = gbufs[b]

                def sc_e(e4, _):
                    e0 = e4 * 4
                    for u in range(4):
                        e = e0 + u
                        cf = plsc.load_gather(coef_b, [_splat(q * CHUNK + e)])
                        for f in range(DHF // 16):
                            gb[e, pl.ds(f * 16, 16)] = (
                                gb[e, pl.ds(f * 16, 16)] * cf)
                    return 0
                lax.fori_loop(0, CHUNK // 4, sc_e, 0)

            for b in range(NB):
                unpack(b, b)
                gath(b, b)
            wait_g(0)
            scale(0, 0)
            scat(0)
            for b in range(1, NB):
                wait_g(b)
                scale(b, b)
                wait_s(b - 1)
                unpack(b + NB - 1, (b + NB - 1) % NB)
                gath(b + NB - 1, (b + NB - 1) % NB)
                scat(b)

            def blk(g, _):
                for b in range(NB):
                    q = g * NB + b
                    pv = (b - 1) % NB
                    wait_g(b)
                    scale(q, b)
                    wait_s(pv)
                    qn = q + NB - 1

                    @pl.when(qn < NCH)
                    def _():
                        unpack(qn, pv)
                        gath(qn, pv)
                    scat(b)
                return 0
            lax.fori_loop(1, NCH // NB, blk, 0)
            wait_s((NCH - 1) % NB)

        plsc.subcore_barrier()

        @pl.when(s < 15)
        def _():
            pltpu.sync_copy(acc_sh.at[pl.ds(base, 632)],
                            scat_hbm.at[c, pl.ds(base, 632)])

        @pl.when(s == 15)
        def _():
            pltpu.sync_copy(acc_sh.at[pl.ds(base, 520)],
                            scat_hbm.at[c, pl.ds(base, 520)])
    return body


_CONV_SCRATCH = [
    pltpu.VMEM((NCH * CHUNK,), jnp.int32),
    pltpu.VMEM((NCH * CHUNK,), jnp.float32),
    pltpu.VMEM((NB, CHUNK), jnp.int32),
    pltpu.VMEM((NB, CHUNK), jnp.int32),
    pltpu.VMEM((NB, CHUNK, DHF), jnp.float32),
    pltpu.VMEM_SHARED((N, DHF), jnp.float32),
    pltpu.SemaphoreType.DMA,
    pltpu.SemaphoreType.DMA,
    pltpu.SemaphoreType.DMA,
    pltpu.SemaphoreType.DMA,
    pltpu.SemaphoreType.DMA,
    pltpu.SemaphoreType.DMA,
]

_conv1_kernel = pl.kernel(
    _conv_body(False),
    out_type=jax.ShapeDtypeStruct((2, N, DIN), jnp.float32),
    mesh=_MESH,
    compiler_params=_SCP,
    scratch_types=_CONV_SCRATCH,
)

_conv2_kernel = pl.kernel(
    _conv_body(True),
    out_type=jax.ShapeDtypeStruct((2, N, DHF), jnp.float32),
    mesh=_MESH,
    compiler_params=_SCP,
    scratch_types=_CONV_SCRATCH,
)


def _stage_b_body(dp_ref, x_ref, g0_ref, b0_ref, h0_ref, dis_ref):
    @pl.when(pl.program_id(0) == 0)
    def _():
        deg = dp_ref[0] + dp_ref[1]
        dis_ref[...] = jnp.where(deg > 0, lax.rsqrt(deg), 0.0)
    h0_ref[...] = x_ref[...] * (g0_ref[...] * _BN0) + b0_ref[...]


def _stage_d_body(sc_ref, w1_ref, b1_ref, g1_ref, bb1_ref, h1_ref):
    agg = sc_ref[0] + sc_ref[1]
    a1 = lax.dot_general(agg, w1_ref[...], (((1,), (0,)), ((), ())),
                         precision=_HI,
                         preferred_element_type=jnp.float32) + b1_ref[...]
    h = jnp.maximum(a1 * (g1_ref[...] * _BN0) + bb1_ref[...], 0.0)
    h1_ref[0] = h[:, :DHF]
    h1_ref[1] = h[:, DHF:]


def _stage_f_body(sc_ref, w2_ref, b2_ref, g2_ref, bb2_ref, batch_ref,
                  wc1_ref, bc1_ref, wc2_ref, bc2_ref, out_ref,
                  pool_acc, cnt_acc):
    i = pl.program_id(0)

    @pl.when(i == 0)
    def _():
        pool_acc[...] = jnp.zeros((NG, DH), jnp.float32)
        cnt_acc[...] = jnp.zeros((NG, 128), jnp.float32)

    agg = jnp.concatenate([sc_ref[0], sc_ref[1]], axis=1)
    a2 = lax.dot_general(agg, w2_ref[...], (((1,), (0,)), ((), ())),
                         precision=_HI,
                         preferred_element_type=jnp.float32) + b2_ref[...]
    h = jnp.maximum(a2 * (g2_ref[...] * _BN0) + bb2_ref[...], 0.0)
    oh = (batch_ref[...] == lax.broadcasted_iota(jnp.int32, (1, NG), 1))
    oh = oh.astype(jnp.float32)
    pool_acc[...] += lax.dot_general(oh, h, (((0,), (0,)), ((), ())),
                                     precision=_HI,
                                     preferred_element_type=jnp.float32)
    cnt_acc[...] += lax.dot_general(oh, jnp.ones((NBLK, 128), jnp.float32),
                                    (((0,), (0,)), ((), ())),
                                    precision=_HI,
                                    preferred_element_type=jnp.float32)

    @pl.when(i == GRID - 1)
    def _():
        cnt = jnp.maximum(cnt_acc[:, 0:1], 1.0)
        pooled = pool_acc[...] / cnt
        z = lax.dot_general(pooled, wc1_ref[...], (((1,), (0,)), ((), ())),
                            precision=_HI,
                            preferred_element_type=jnp.float32) + bc1_ref[...]
        z = jnp.maximum(z, 0.0)
        out_ref[...] = lax.dot_general(z, wc2_ref[...],
                                       (((1,), (0,)), ((), ())),
                                       precision=_HI,
                                       preferred_element_type=jnp.float32
                                       ) + bc2_ref[...]


def _rows(shape):
    return pl.BlockSpec(shape, lambda i: (i,) + (0,) * (len(shape) - 1))


def _const(shape):
    return pl.BlockSpec(shape, lambda i: (0,) * len(shape))


_stage_b = pl.pallas_call(
    _stage_b_body,
    grid=(GRID,),
    in_specs=[_const((2, NROW, 128)), _rows((NBLK, DIN)),
              _const((1, DIN)), _const((1, DIN))],
    out_specs=[_rows((NBLK, DIN)), _const((NROW, 128))],
    out_shape=[jax.ShapeDtypeStruct((N, DIN), jnp.float32),
               jax.ShapeDtypeStruct((NROW, 128), jnp.float32)],
)

_stage_d = pl.pallas_call(
    _stage_d_body,
    grid=(GRID,),
    in_specs=[pl.BlockSpec((2, NBLK, DIN), lambda i: (0, i, 0)),
              _const((DIN, DH)), _const((1, DH)),
              _const((1, DH)), _const((1, DH))],
    out_specs=pl.BlockSpec((2, NBLK, DHF), lambda i: (0, i, 0)),
    out_shape=jax.ShapeDtypeStruct((2, N, DHF), jnp.float32),
)

_stage_f = pl.pallas_call(
    _stage_f_body,
    grid=(GRID,),
    in_specs=[pl.BlockSpec((2, NBLK, DHF), lambda i: (0, i, 0)),
              _const((DH, DH)), _const((1, DH)),
              _const((1, DH)), _const((1, DH)),
              _rows((NBLK, 1)),
              _const((DH, DH)), _const((1, DH)),
              _const((DH, 2)), _const((1, 2))],
    out_specs=_const((NG, 2)),
    out_shape=jax.ShapeDtypeStruct((NG, 2), jnp.float32),
    scratch_shapes=[pltpu.VMEM((NG, DH), jnp.float32),
                    pltpu.VMEM((NG, 128), jnp.float32)],
)


def kernel(x, edge_index, batch, edge_attr, bn0_g, bn0_b, W1, b1, bn1_g,
           bn1_b, W2, b2, bn2_g, bn2_b, Wc1, bc1, Wc2, bc2):
    sl = jnp.arange(N, dtype=jnp.int32)
    row = jnp.concatenate([edge_index[0].astype(jnp.int32), sl])
    col = jnp.concatenate([edge_index[1].astype(jnp.int32), sl])
    ew = jnp.concatenate([edge_attr, jnp.ones((N,), jnp.float32)])
    packed = jnp.bitwise_or(row, jnp.left_shift(col, 14))
    pad = E2P - packed.shape[0]
    packed = jnp.pad(packed, (0, pad)).reshape(32, NCH, CHUNK)
    ew = jnp.pad(ew, (0, pad)).reshape(32, NCH, CHUNK)
    batch2 = batch.astype(jnp.int32).reshape(N, 1)
    r1 = lambda a: a.reshape(1, -1)

    deg_parts = _deg_kernel(packed, ew)
    h0, dis = _stage_b(deg_parts, x, r1(bn0_g), r1(bn0_b))
    coef = _coef_kernel(dis.reshape(NPAD), packed, ew)
    packed2 = packed.reshape(32, NCH * CHUNK)
    scat1 = _conv1_kernel(h0, packed2, coef.reshape(32, NCH * CHUNK))
    h1 = _stage_d(scat1, W1, r1(b1), r1(bn1_g), r1(bn1_b))
    scat2 = _conv2_kernel(h1.reshape(2 * N, DHF), packed2,
                          coef.reshape(32, NCH * CHUNK))
    return _stage_f(scat2, W2, r1(b2), r1(bn2_g), r1(bn2_b), batch2,
                    Wc1, r1(bc1), Wc2, r1(bc2))
